# baseline (speedup 1.0000x reference)
import jax
import jax.numpy as jnp
from jax.experimental import pallas as pl

K = 1024
D = 32
BLOCK = 512


def _vq_kernel(x_ref, emb_ref, q_ref, c_ref, l_ref):
    x = x_ref[...]
    e = emb_ref[...]
    e2 = jnp.sum(e * e, axis=1)
    x2 = jnp.sum(x * x, axis=1)
    xe = jax.lax.dot_general(x, e, (((1,), (1,)), ((), ())),
                             preferred_element_type=jnp.float32)
    d = x2[:, None] + e2[None, :] - 2.0 * xe
    m = jnp.min(d, axis=1)
    iota = jax.lax.broadcasted_iota(jnp.int32, (BLOCK, K), 1)
    c = jnp.min(jnp.where(d <= m[:, None], iota, K), axis=1).astype(jnp.int32)
    onehot = (iota == c[:, None]).astype(jnp.float32)
    q = jnp.dot(onehot, e, preferred_element_type=jnp.float32)
    q_ref[...] = q
    c_ref[...] = c.reshape(1, 1, BLOCK)
    l_ref[...] = (m * (0.2 / D)).reshape(1, 1, BLOCK)


def kernel(h, emb):
    flat = h.reshape(-1, D)
    n = flat.shape[0]
    nb = n // BLOCK
    q, c, l = pl.pallas_call(
        _vq_kernel,
        grid=(nb,),
        in_specs=[
            pl.BlockSpec((BLOCK, D), lambda i: (i, 0)),
            pl.BlockSpec((K, D), lambda i: (0, 0)),
        ],
        out_specs=[
            pl.BlockSpec((BLOCK, D), lambda i: (i, 0)),
            pl.BlockSpec((1, 1, BLOCK), lambda i: (i, 0, 0)),
            pl.BlockSpec((1, 1, BLOCK), lambda i: (i, 0, 0)),
        ],
        out_shape=[
            jax.ShapeDtypeStruct((n, D), jnp.float32),
            jax.ShapeDtypeStruct((nb, 1, BLOCK), jnp.int32),
            jax.ShapeDtypeStruct((nb, 1, BLOCK), jnp.float32),
        ],
    )(flat, emb)
    quantized = q.reshape(h.shape)
    return quantized, c.reshape(n, 1), l.reshape(n)
